# baseline (speedup 1.0000x reference)
import functools

import jax
import jax.numpy as jnp
from jax import lax
from jax.experimental import pallas as pl
from jax.experimental.pallas import tpu as pltpu
from jax.experimental.pallas import tpu_sc as plsc

_B = 4
_C = 3
_HW = 4096
_T = _HW * _C
_DIM = 768
_VOCAB = 1024
_NC = 2
_NS = 16
_NW = _NC * _NS
_PPW = _HW // _NW
_TPW = _PPW * _C
_CHUNK = 24
_NCHUNK = _TPW // _CHUNK


def _sc_body(x_hbm, tab_hbm, pos_hbm, cls_hbm, out_hbm,
             cls_v, xbuf, idx_all, pb0, pb1, g0, g1, g2, g3,
             sg0, sg1, sg2, sg3, st0, st1, st2, st3, sp0, sp1, sx):
    wid = lax.axis_index("s") * _NC + lax.axis_index("c")
    p0 = wid * _PPW
    t0 = wid * _TPW

    gbufs = [g0, g1, g2, g3]
    sgs = [sg0, sg1, sg2, sg3]
    sts = [st0, st1, st2, st3]
    pbufs = [pb0, pb1]
    sps = [sp0, sp1]

    def gather_fire(c, b):
        idx_sl = idx_all.at[pl.ds(b * _TPW + c * _CHUNK, _CHUNK)]
        pltpu.async_copy(tab_hbm.at[idx_sl], gbufs[b], sgs[b])

    def gather_wait(c, b):
        idx_sl = idx_all.at[pl.ds(b * _TPW + c * _CHUNK, _CHUNK)]
        pltpu.make_async_copy(tab_hbm.at[idx_sl], gbufs[b], sgs[b]).wait()

    def store_fire(c, b):
        dst = out_hbm.at[pl.ds(b * _T + t0 + c * _CHUNK, _CHUNK)]
        pltpu.async_copy(gbufs[b], dst, sts[b])

    def store_wait(b):
        dst = out_hbm.at[pl.ds(0, _CHUNK)]
        pltpu.make_async_copy(gbufs[b], dst, sts[b]).wait()

    def pos_fire(c, par):
        src = pos_hbm.at[pl.ds(t0 + c * _CHUNK, _CHUNK)]
        pltpu.async_copy(src, pbufs[par], sps[par])

    def pos_wait(par):
        src = pos_hbm.at[pl.ds(0, _CHUNK)]
        pltpu.make_async_copy(src, pbufs[par], sps[par]).wait()

    def add_pos(b, par):
        g = gbufs[b]
        pb = pbufs[par]

        @plsc.parallel_loop(0, _CHUNK, step=1, unroll=4)
        def row_step(r):
            for u in range(_DIM // 16):
                plsc.addupdate(g.at[r, pl.ds(u * 16, 16)],
                               pb[r, pl.ds(u * 16, 16)])

    for r in range(_B * _C):
        pltpu.async_copy(x_hbm.at[r, pl.ds(p0, _PPW)], xbuf.at[r], sx)
    pos_fire(0, 0)
    pos_fire(1, 1)
    pltpu.sync_copy(cls_hbm, cls_v)
    for r in range(_B * _C):
        pltpu.make_async_copy(x_hbm.at[r, pl.ds(p0, _PPW)], xbuf.at[r],
                              sx).wait()

    iota = lax.iota(jnp.int32, 16)

    for b in range(_B):
        for ch in range(_C):
            def idx_step(j, carry, b=b, ch=ch):
                xv = xbuf[b * _C + ch, pl.ds(j * 16, 16)]
                k = jnp.clip((xv * 1023.0 + 0.5).astype(jnp.int32), 0, 1023)
                km = jnp.maximum(k - 1, 0)
                kp = jnp.minimum(k + 1, 1023)
                c0 = plsc.load_gather(cls_v, [km])
                c1 = plsc.load_gather(cls_v, [k])
                c2 = plsc.load_gather(cls_v, [kp])
                d0 = (xv - c0) * (xv - c0)
                d1 = (xv - c1) * (xv - c1)
                d2 = (xv - c2) * (xv - c2)
                bi = km
                bd = d0
                s1 = d1 < bd
                bi = jnp.where(s1, k, bi)
                bd = jnp.where(s1, d1, bd)
                s2 = d2 < bd
                bi = jnp.where(s2, kp, bi)
                tloc = (iota + j * 16) * _C + (ch + b * _TPW)
                plsc.store_scatter(idx_all, [tloc], bi)
                return carry

            lax.fori_loop(0, _PPW // 16, idx_step, 0)
        if b == 0:
            gather_fire(0, 0)
        if b == 1:
            gather_fire(0, 1)

    def do_step(c, b, par):
        gather_wait(c, b)
        if b == 0:
            pos_wait(par)
        if b < 2:
            @pl.when(c >= 1)
            def _():
                store_wait(b + 2)

            gather_fire(c, b + 2)
        else:
            @pl.when(c <= _NCHUNK - 2)
            def _():
                store_wait(b - 2)
                gather_fire(c + 1, b - 2)

        add_pos(b, par)
        store_fire(c, b)
        if b == 3:
            @pl.when(c <= _NCHUNK - 3)
            def _():
                pos_fire(c + 2, par)

    def pair_step(i, carry):
        cc = i * 2
        for j in range(2):
            c = cc + j
            for b in range(_B):
                do_step(c, b, j)
        return carry

    lax.fori_loop(0, _NCHUNK // 2, pair_step, 0)

    for b in range(_B):
        store_wait(b)


def kernel(x, embed_table, pos_embed, classes):
    x2 = x.reshape(_B * _C, _HW)
    pos2 = pos_embed.reshape(_T, _DIM)
    cls1 = classes.reshape(_VOCAB)

    mesh = plsc.VectorSubcoreMesh(core_axis_name="c", subcore_axis_name="s")
    f = pl.kernel(
        _sc_body,
        out_type=jax.ShapeDtypeStruct((_B * _T, _DIM), jnp.float32),
        mesh=mesh,
        compiler_params=pltpu.CompilerParams(needs_layout_passes=False),
        scratch_types=[
            pltpu.VMEM((_VOCAB,), jnp.float32),
            pltpu.VMEM((_B * _C, _PPW), jnp.float32),
            pltpu.VMEM((_B * _TPW,), jnp.int32),
            pltpu.VMEM((_CHUNK, _DIM), jnp.float32),
            pltpu.VMEM((_CHUNK, _DIM), jnp.float32),
            pltpu.VMEM((_CHUNK, _DIM), jnp.float32),
            pltpu.VMEM((_CHUNK, _DIM), jnp.float32),
            pltpu.VMEM((_CHUNK, _DIM), jnp.float32),
            pltpu.VMEM((_CHUNK, _DIM), jnp.float32),
            pltpu.SemaphoreType.DMA,
            pltpu.SemaphoreType.DMA,
            pltpu.SemaphoreType.DMA,
            pltpu.SemaphoreType.DMA,
            pltpu.SemaphoreType.DMA,
            pltpu.SemaphoreType.DMA,
            pltpu.SemaphoreType.DMA,
            pltpu.SemaphoreType.DMA,
            pltpu.SemaphoreType.DMA,
            pltpu.SemaphoreType.DMA,
            pltpu.SemaphoreType.DMA,
        ],
    )
    out = f(x2, embed_table, pos2, cls1)
    return out.reshape(_B, _T, _DIM)


# speedup vs baseline: 1.2044x; 1.2044x over previous
import functools

import jax
import jax.numpy as jnp
from jax import lax
from jax.experimental import pallas as pl
from jax.experimental.pallas import tpu as pltpu
from jax.experimental.pallas import tpu_sc as plsc

_B = 4
_C = 3
_HW = 4096
_T = _HW * _C
_DIM = 768
_VOCAB = 1024
_NC = 2
_NS = 16
_NW = _NC * _NS
_PPW = _HW // _NW
_TPW = _PPW * _C
_CHUNK = 24
_NCHUNK = _TPW // _CHUNK


def _sc_body(x_hbm, tab_hbm, pos_hbm, cls_hbm, out_hbm,
             cls_v, xbuf, idx_all, pb0, pb1, g0, g1, g2, g3,
             sg0, sg1, sg2, sg3, st0, st1, st2, st3, sp0, sp1, sx):
    wid = lax.axis_index("s") * _NC + lax.axis_index("c")
    p0 = wid * _PPW
    t0 = wid * _TPW

    gbufs = [g0, g1, g2, g3]
    sgs = [sg0, sg1, sg2, sg3]
    sts = [st0, st1, st2, st3]
    pbufs = [pb0, pb1]
    sps = [sp0, sp1]

    def gather_fire(c, b):
        idx_sl = idx_all.at[pl.ds(b * _TPW + c * _CHUNK, _CHUNK)]
        pltpu.async_copy(tab_hbm.at[idx_sl], gbufs[b], sgs[b])

    def gather_wait(c, b):
        idx_sl = idx_all.at[pl.ds(b * _TPW + c * _CHUNK, _CHUNK)]
        pltpu.make_async_copy(tab_hbm.at[idx_sl], gbufs[b], sgs[b]).wait()

    def store_fire(c, b):
        dst = out_hbm.at[pl.ds(b * _T + t0 + c * _CHUNK, _CHUNK)]
        pltpu.async_copy(gbufs[b], dst, sts[b])

    def store_wait(b):
        dst = out_hbm.at[pl.ds(0, _CHUNK)]
        pltpu.make_async_copy(gbufs[b], dst, sts[b]).wait()

    def pos_fire(c, par):
        src = pos_hbm.at[pl.ds(t0 + c * _CHUNK, _CHUNK)]
        pltpu.async_copy(src, pbufs[par], sps[par])

    def pos_wait(par):
        src = pos_hbm.at[pl.ds(0, _CHUNK)]
        pltpu.make_async_copy(src, pbufs[par], sps[par]).wait()

    def add_pos(b, par):
        g = gbufs[b]
        pb = pbufs[par]

        @plsc.parallel_loop(0, _CHUNK, step=1, unroll=2)
        def row_step(r):
            for u in range(_DIM // 16):
                plsc.addupdate(g.at[r, pl.ds(u * 16, 16)],
                               pb[r, pl.ds(u * 16, 16)])

    for r in range(_B * _C):
        pltpu.async_copy(x_hbm.at[r, pl.ds(p0, _PPW)], xbuf.at[r], sx)
    pos_fire(0, 0)
    pos_fire(1, 1)
    pltpu.sync_copy(cls_hbm, cls_v)
    for r in range(_B * _C):
        pltpu.make_async_copy(x_hbm.at[r, pl.ds(p0, _PPW)], xbuf.at[r],
                              sx).wait()

    iota = lax.iota(jnp.int32, 16)

    for b in range(_B):
        for ch in range(_C):
            def idx_step(j, carry, b=b, ch=ch):
                xv = xbuf[b * _C + ch, pl.ds(j * 16, 16)]
                k = jnp.clip((xv * 1023.0 + 0.5).astype(jnp.int32), 0, 1023)
                km = jnp.maximum(k - 1, 0)
                kp = jnp.minimum(k + 1, 1023)
                c0 = plsc.load_gather(cls_v, [km])
                c1 = plsc.load_gather(cls_v, [k])
                c2 = plsc.load_gather(cls_v, [kp])
                d0 = (xv - c0) * (xv - c0)
                d1 = (xv - c1) * (xv - c1)
                d2 = (xv - c2) * (xv - c2)
                bi = km
                bd = d0
                s1 = d1 < bd
                bi = jnp.where(s1, k, bi)
                bd = jnp.where(s1, d1, bd)
                s2 = d2 < bd
                bi = jnp.where(s2, kp, bi)
                tloc = (iota + j * 16) * _C + (ch + b * _TPW)
                plsc.store_scatter(idx_all, [tloc], bi)
                return carry

            lax.fori_loop(0, _PPW // 16, idx_step, 0)
        if b == 0:
            gather_fire(0, 0)
        if b == 1:
            gather_fire(0, 1)

    def do_step(c, b, par):
        gather_wait(c, b)
        if b == 0:
            pos_wait(par)
        if b < 2:
            @pl.when(c >= 1)
            def _():
                store_wait(b + 2)

            gather_fire(c, b + 2)
        else:
            @pl.when(c <= _NCHUNK - 2)
            def _():
                store_wait(b - 2)
                gather_fire(c + 1, b - 2)

        add_pos(b, par)
        store_fire(c, b)
        if b == 3:
            @pl.when(c <= _NCHUNK - 3)
            def _():
                pos_fire(c + 2, par)

    def pair_step(i, carry):
        cc = i * 2
        for j in range(2):
            c = cc + j
            for b in range(_B):
                do_step(c, b, j)
        return carry

    lax.fori_loop(0, _NCHUNK // 2, pair_step, 0)

    for b in range(_B):
        store_wait(b)


def kernel(x, embed_table, pos_embed, classes):
    x2 = x.reshape(_B * _C, _HW)
    pos2 = pos_embed.reshape(_T, _DIM)
    cls1 = classes.reshape(_VOCAB)

    mesh = plsc.VectorSubcoreMesh(core_axis_name="c", subcore_axis_name="s")
    f = pl.kernel(
        _sc_body,
        out_type=jax.ShapeDtypeStruct((_B * _T, _DIM), jnp.float32),
        mesh=mesh,
        compiler_params=pltpu.CompilerParams(needs_layout_passes=False),
        scratch_types=[
            pltpu.VMEM((_VOCAB,), jnp.float32),
            pltpu.VMEM((_B * _C, _PPW), jnp.float32),
            pltpu.VMEM((_B * _TPW,), jnp.int32),
            pltpu.VMEM((_CHUNK, _DIM), jnp.float32),
            pltpu.VMEM((_CHUNK, _DIM), jnp.float32),
            pltpu.VMEM((_CHUNK, _DIM), jnp.float32),
            pltpu.VMEM((_CHUNK, _DIM), jnp.float32),
            pltpu.VMEM((_CHUNK, _DIM), jnp.float32),
            pltpu.VMEM((_CHUNK, _DIM), jnp.float32),
            pltpu.SemaphoreType.DMA,
            pltpu.SemaphoreType.DMA,
            pltpu.SemaphoreType.DMA,
            pltpu.SemaphoreType.DMA,
            pltpu.SemaphoreType.DMA,
            pltpu.SemaphoreType.DMA,
            pltpu.SemaphoreType.DMA,
            pltpu.SemaphoreType.DMA,
            pltpu.SemaphoreType.DMA,
            pltpu.SemaphoreType.DMA,
            pltpu.SemaphoreType.DMA,
        ],
    )
    out = f(x2, embed_table, pos2, cls1)
    return out.reshape(_B, _T, _DIM)
